# Initial kernel scaffold; baseline (speedup 1.0000x reference)
import jax
import jax.numpy as jnp
from jax.experimental import pallas as pl


def kernel(node_attr, connectivity, edge_attr, u, W_e, b_e, W_v, b_v, W_u, b_u):
    raise NotImplementedError("write your pallas kernel here")


# speedup vs baseline: 3.5278x; 3.5278x over previous
import functools

import jax
import jax.numpy as jnp
from jax import lax
from jax.experimental import pallas as pl
from jax.experimental.pallas import tpu as pltpu
from jax.experimental.pallas import tpu_sc as plsc

N = 10000
E = 320000
D = 128
DE = 16
DU = 32

NC = 2
NS = 16
NW = NC * NS
CH = 128
NCHUNK = E // CH
N_PAD = 10240
ROWS_PER_TILE = N_PAD // NS


_sc_mesh = plsc.VectorSubcoreMesh(
    core_axis_name="c", subcore_axis_name="s", num_cores=NC, num_subcores=NS
)


@functools.partial(
    pl.kernel,
    out_type=(
        jax.ShapeDtypeStruct((E, DE), jnp.float32),
        jax.ShapeDtypeStruct((NC, N_PAD, DE), jnp.float32),
        jax.ShapeDtypeStruct((NW, DE), jnp.float32),
    ),
    mesh=_sc_mesh,
    scratch_types=[
        pltpu.VMEM((CH,), jnp.int32),
        pltpu.VMEM((CH,), jnp.int32),
        pltpu.VMEM((CH, DE), jnp.float32),
        pltpu.VMEM((CH, DE), jnp.float32),
        pltpu.VMEM((CH, DE), jnp.float32),
        pltpu.VMEM((CH, DE), jnp.float32),
        pltpu.VMEM((ROWS_PER_TILE, DE), jnp.float32),
        pltpu.VMEM_SHARED((N_PAD, DE), jnp.float32),
        pltpu.SemaphoreType.DMA,
        pltpu.SemaphoreType.DMA,
    ],
    compiler_params=pltpu.CompilerParams(use_tc_tiling_on_sc=False),
)
def _edge_sc(row_hbm, col_hbm, et_hbm, psrc_hbm, pdst_hbm,
             eout_hbm, aggp_hbm, esum_hbm,
             idxr_v, idxc_v, et_v, src_v, dst_v, eo_v, z_v, agg_sh,
             sem1, sem2):
    cid = lax.axis_index("c")
    sid = lax.axis_index("s")
    wid = sid * NC + cid

    def zero_body(i, _):
        z_v[i] = jnp.zeros((DE,), jnp.float32)
        return 0
    lax.fori_loop(0, ROWS_PER_TILE, zero_body, 0)
    pltpu.sync_copy(z_v, agg_sh.at[pl.ds(sid * ROWS_PER_TILE, ROWS_PER_TILE)])
    plsc.subcore_barrier()

    base_chunks = NCHUNK // NW
    nloc = jnp.where(wid < NCHUNK - base_chunks * NW, base_chunks + 1,
                     base_chunks)

    def chunk_body(j, vsum):
        base = (wid + j * NW) * CH
        pltpu.sync_copy(row_hbm.at[pl.ds(base, CH)], idxr_v)
        pltpu.sync_copy(col_hbm.at[pl.ds(base, CH)], idxc_v)
        pltpu.sync_copy(et_hbm.at[pl.ds(base, CH)], et_v)
        cp1 = pltpu.async_copy(psrc_hbm.at[idxr_v], src_v, sem1)
        cp2 = pltpu.async_copy(pdst_hbm.at[idxc_v], dst_v, sem2)
        cp1.wait()
        cp2.wait()

        def edge_body(i, vs):
            val = jnp.maximum(src_v[i] + dst_v[i] + et_v[i], 0.0)
            eo_v[i] = val
            return vs + val
        vsum = lax.fori_loop(0, CH, edge_body, vsum)

        pltpu.sync_copy(eo_v, eout_hbm.at[pl.ds(base, CH)])
        pltpu.sync_copy(eo_v, agg_sh.at[idxc_v], add=True)
        return vsum

    vsum = lax.fori_loop(0, nloc, chunk_body, jnp.zeros((DE,), jnp.float32))

    z_v[0] = vsum
    pltpu.sync_copy(z_v.at[0], esum_hbm.at[wid])

    plsc.subcore_barrier()
    pltpu.sync_copy(
        agg_sh.at[pl.ds(sid * ROWS_PER_TILE, ROWS_PER_TILE)],
        aggp_hbm.at[cid, pl.ds(sid * ROWS_PER_TILE, ROWS_PER_TILE)],
    )


def _proj_body(node_ref, wsrc_ref, wdst_ref, psrc_ref, pdst_ref):
    x = node_ref[...]
    psrc_ref[...] = jnp.dot(x, wsrc_ref[...], preferred_element_type=jnp.float32)
    pdst_ref[...] = jnp.dot(x, wdst_ref[...], preferred_element_type=jnp.float32)


def _edge_term_body(ea_ref, w_ref, b_ref, out_ref):
    out_ref[...] = (
        jnp.dot(ea_ref[...], w_ref[...], preferred_element_type=jnp.float32)
        + b_ref[...]
    )


def _node_body(node_ref, aggp_ref, wv1_ref, wv2_ref, bv_ref, vout_ref, vsum_ref):
    i = pl.program_id(0)
    agg = aggp_ref[0] + aggp_ref[1]
    v = jnp.dot(node_ref[...], wv1_ref[...], preferred_element_type=jnp.float32)
    v += jnp.dot(agg, wv2_ref[...], preferred_element_type=jnp.float32)
    v = jnp.maximum(v + bv_ref[...], 0.0)
    vout_ref[...] = v

    @pl.when(i == 0)
    def _():
        vsum_ref[...] = jnp.zeros_like(vsum_ref)
    vsum_ref[...] += jnp.sum(v, axis=0, keepdims=True)


def _global_body(vsum_ref, esum_ref, u_ref, wuv_ref, wue_ref, wuu_ref, bu_ref,
                 out_ref):
    mean_v = vsum_ref[...] / N
    mean_e = jnp.sum(esum_ref[...], axis=0, keepdims=True) / E
    acc = jnp.dot(mean_v, wuv_ref[...], preferred_element_type=jnp.float32)
    acc += jnp.dot(mean_e, wue_ref[...], preferred_element_type=jnp.float32)
    acc += jnp.dot(u_ref[...], wuu_ref[...], preferred_element_type=jnp.float32)
    out_ref[...] = jnp.maximum(acc + bu_ref[...], 0.0)


def kernel(node_attr, connectivity, edge_attr, u, W_e, b_e, W_v, b_v, W_u, b_u):
    row = connectivity[0]
    col = connectivity[1]
    W_src = W_e[:D]
    W_dst = W_e[D:2 * D]
    W_eg = W_e[2 * D:]
    W_v1 = W_v[:D]
    W_v2 = W_v[D:]
    W_uv = W_u[:D]
    W_ue = W_u[D:D + DE]
    W_uu = W_u[D + DE:]

    psrc, pdst = pl.pallas_call(
        _proj_body,
        out_shape=(
            jax.ShapeDtypeStruct((N, DE), jnp.float32),
            jax.ShapeDtypeStruct((N, DE), jnp.float32),
        ),
    )(node_attr, W_src, W_dst)

    EB = 8000
    edge_term = pl.pallas_call(
        _edge_term_body,
        grid=(E // EB,),
        in_specs=[
            pl.BlockSpec((EB, DE), lambda i: (i, 0)),
            pl.BlockSpec((DE, DE), lambda i: (0, 0)),
            pl.BlockSpec((1, DE), lambda i: (0, 0)),
        ],
        out_specs=pl.BlockSpec((EB, DE), lambda i: (i, 0)),
        out_shape=jax.ShapeDtypeStruct((E, DE), jnp.float32),
    )(edge_attr, W_eg, b_e.reshape(1, DE))

    e_out, aggp, esum = _edge_sc(row, col, edge_term, psrc, pdst)

    NB = 2000
    v_out, vsum = pl.pallas_call(
        _node_body,
        grid=(N // NB,),
        in_specs=[
            pl.BlockSpec((NB, D), lambda i: (i, 0)),
            pl.BlockSpec((NC, NB, DE), lambda i: (0, i, 0)),
            pl.BlockSpec((D, D), lambda i: (0, 0)),
            pl.BlockSpec((DE, D), lambda i: (0, 0)),
            pl.BlockSpec((1, D), lambda i: (0, 0)),
        ],
        out_specs=(
            pl.BlockSpec((NB, D), lambda i: (i, 0)),
            pl.BlockSpec((1, D), lambda i: (0, 0)),
        ),
        out_shape=(
            jax.ShapeDtypeStruct((N, D), jnp.float32),
            jax.ShapeDtypeStruct((1, D), jnp.float32),
        ),
    )(node_attr, aggp, W_v1, W_v2, b_v.reshape(1, D))

    u_out = pl.pallas_call(
        _global_body,
        out_shape=jax.ShapeDtypeStruct((1, DU), jnp.float32),
    )(vsum, esum, u, W_uv, W_ue, W_uu, b_u.reshape(1, DU))

    return (v_out, e_out, u_out)


# speedup vs baseline: 5.9275x; 1.6802x over previous
import functools

import jax
import jax.numpy as jnp
from jax import lax
from jax.experimental import pallas as pl
from jax.experimental.pallas import tpu as pltpu
from jax.experimental.pallas import tpu_sc as plsc

N = 10000
E = 320000
D = 128
DE = 16
DU = 32

NC = 2
NS = 16
NW = NC * NS
CH = 640
SUB = 128
NSUB = CH // SUB
NCHUNK = E // CH
EP = E // 8
CHP = CH // 8
N_PAD = 10240
ROWS_PER_TILE = N_PAD // NS


_sc_mesh = plsc.VectorSubcoreMesh(
    core_axis_name="c", subcore_axis_name="s", num_cores=NC, num_subcores=NS
)


@functools.partial(
    pl.kernel,
    out_type=(
        jax.ShapeDtypeStruct((EP, 128), jnp.float32),
        jax.ShapeDtypeStruct((NC, N_PAD, DE), jnp.float32),
    ),
    mesh=_sc_mesh,
    scratch_types=[
        pltpu.VMEM((NSUB, SUB), jnp.int32),
        pltpu.VMEM((NSUB, SUB), jnp.int32),
        pltpu.VMEM((CHP, 128), jnp.float32),
        pltpu.VMEM((CH, DE), jnp.float32),
        pltpu.VMEM((CH, DE), jnp.float32),
        pltpu.VMEM((CH, DE), jnp.float32),
        pltpu.VMEM((CHP, 128), jnp.float32),
        pltpu.VMEM((ROWS_PER_TILE, DE), jnp.float32),
        pltpu.VMEM_SHARED((N_PAD, DE), jnp.float32),
        pltpu.SemaphoreType.DMA,
        pltpu.SemaphoreType.DMA,
    ],
    compiler_params=pltpu.CompilerParams(use_tc_tiling_on_sc=False),
)
def _edge_sc(row_hbm, col_hbm, et_hbm, psrc_hbm, pdst_hbm,
             eout_hbm, aggp_hbm,
             idxr_v, idxc_v, et_v, src_v, dst_v, eo_v, eo40_v, z_v, agg_sh,
             sem1, sem2):
    cid = lax.axis_index("c")
    sid = lax.axis_index("s")
    wid = sid * NC + cid

    def zero_body(i, _):
        z_v[i] = jnp.zeros((DE,), jnp.float32)
        return 0
    lax.fori_loop(0, ROWS_PER_TILE, zero_body, 0)
    pltpu.sync_copy(z_v, agg_sh.at[pl.ds(sid * ROWS_PER_TILE, ROWS_PER_TILE)])
    plsc.subcore_barrier()

    base_chunks = NCHUNK // NW
    nloc = jnp.where(wid < NCHUNK - base_chunks * NW, base_chunks + 1,
                     base_chunks)

    def chunk_body(j, _):
        chunk = wid + j * NW
        cpr = pltpu.async_copy(row_hbm.at[pl.ds(chunk * NSUB, NSUB)], idxr_v,
                               sem2)
        cpc = pltpu.async_copy(col_hbm.at[pl.ds(chunk * NSUB, NSUB)], idxc_v,
                               sem2)
        cpe = pltpu.async_copy(et_hbm.at[pl.ds(chunk * CHP, CHP)], et_v, sem2)
        cpr.wait()
        cpc.wait()
        gathers = []
        for k in range(NSUB):
            gathers.append(pltpu.async_copy(
                psrc_hbm.at[idxr_v.at[k]], src_v.at[pl.ds(k * SUB, SUB)],
                sem1))
            gathers.append(pltpu.async_copy(
                pdst_hbm.at[idxc_v.at[k]], dst_v.at[pl.ds(k * SUB, SUB)],
                sem1))
        for g in gathers:
            g.wait()
        cpe.wait()

        def edge_body(i, _):
            t = i // 8
            o = (i % 8) * DE
            val = jnp.maximum(src_v[i] + dst_v[i] + et_v[t, pl.ds(o, DE)], 0.0)
            eo_v[i] = val
            eo40_v[t, pl.ds(o, DE)] = val
            return 0
        lax.fori_loop(0, CH, edge_body, 0)

        pltpu.sync_copy(eo40_v, eout_hbm.at[pl.ds(chunk * CHP, CHP)])
        for k in range(NSUB):
            pltpu.sync_copy(
                eo_v.at[pl.ds(k * SUB, SUB)],
                agg_sh.at[idxc_v.at[k]], add=True)
        return 0

    lax.fori_loop(0, nloc, chunk_body, 0)

    plsc.subcore_barrier()
    pltpu.sync_copy(
        agg_sh.at[pl.ds(sid * ROWS_PER_TILE, ROWS_PER_TILE)],
        aggp_hbm.at[cid, pl.ds(sid * ROWS_PER_TILE, ROWS_PER_TILE)],
    )


def _proj_body(node_ref, wsrc_ref, wdst_ref, psrc_ref, pdst_ref):
    x = node_ref[...]
    psrc_ref[...] = jnp.dot(x, wsrc_ref[...], preferred_element_type=jnp.float32)
    pdst_ref[...] = jnp.dot(x, wdst_ref[...], preferred_element_type=jnp.float32)


def _edge_term_body(ea_ref, w_ref, b_ref, out_ref):
    out_ref[...] = (
        jnp.dot(ea_ref[...], w_ref[...], preferred_element_type=jnp.float32)
        + b_ref[...]
    )


def _node_body(node_ref, aggp_ref, wv1_ref, wv2_ref, bv_ref,
               vout_ref, vsum_ref, esum_ref):
    i = pl.program_id(0)
    agg = aggp_ref[0] + aggp_ref[1]
    v = jnp.dot(node_ref[...], wv1_ref[...], preferred_element_type=jnp.float32)
    v += jnp.dot(agg, wv2_ref[...], preferred_element_type=jnp.float32)
    v = jnp.maximum(v + bv_ref[...], 0.0)
    vout_ref[...] = v

    @pl.when(i == 0)
    def _():
        vsum_ref[...] = jnp.zeros_like(vsum_ref)
        esum_ref[...] = jnp.zeros_like(esum_ref)
    vsum_ref[...] += jnp.sum(v, axis=0, keepdims=True)
    esum_ref[...] += jnp.sum(agg, axis=0, keepdims=True)


def _global_body(vsum_ref, esum_ref, u_ref, wuv_ref, wue_ref, wuu_ref, bu_ref,
                 out_ref):
    mean_v = vsum_ref[...] / N
    mean_e = esum_ref[...] / E
    acc = jnp.dot(mean_v, wuv_ref[...], preferred_element_type=jnp.float32)
    acc += jnp.dot(mean_e, wue_ref[...], preferred_element_type=jnp.float32)
    acc += jnp.dot(u_ref[...], wuu_ref[...], preferred_element_type=jnp.float32)
    out_ref[...] = jnp.maximum(acc + bu_ref[...], 0.0)


def kernel(node_attr, connectivity, edge_attr, u, W_e, b_e, W_v, b_v, W_u, b_u):
    row = connectivity[0]
    col = connectivity[1]
    row2d = row.reshape(E // SUB, SUB)
    col2d = col.reshape(E // SUB, SUB)
    W_src = W_e[:D]
    W_dst = W_e[D:2 * D]
    W_eg = W_e[2 * D:]
    W_v1 = W_v[:D]
    W_v2 = W_v[D:]
    W_uv = W_u[:D]
    W_ue = W_u[D:D + DE]
    W_uu = W_u[D + DE:]

    psrc, pdst = pl.pallas_call(
        _proj_body,
        out_shape=(
            jax.ShapeDtypeStruct((N, DE), jnp.float32),
            jax.ShapeDtypeStruct((N, DE), jnp.float32),
        ),
    )(node_attr, W_src, W_dst)

    ea40 = edge_attr.reshape(EP, 128)
    W_blk = jnp.kron(jnp.eye(8, dtype=jnp.float32), W_eg)
    b_blk = jnp.tile(b_e, 8).reshape(1, 128)
    EBP = 2000
    edge_term = pl.pallas_call(
        _edge_term_body,
        grid=(EP // EBP,),
        in_specs=[
            pl.BlockSpec((EBP, 128), lambda i: (i, 0)),
            pl.BlockSpec((128, 128), lambda i: (0, 0)),
            pl.BlockSpec((1, 128), lambda i: (0, 0)),
        ],
        out_specs=pl.BlockSpec((EBP, 128), lambda i: (i, 0)),
        out_shape=jax.ShapeDtypeStruct((EP, 128), jnp.float32),
    )(ea40, W_blk, b_blk)

    eout_packed, aggp = _edge_sc(row2d, col2d, edge_term, psrc, pdst)
    e_out = eout_packed.reshape(E, DE)

    NB = 2000
    v_out, vsum, esum = pl.pallas_call(
        _node_body,
        grid=(N // NB,),
        in_specs=[
            pl.BlockSpec((NB, D), lambda i: (i, 0)),
            pl.BlockSpec((NC, NB, DE), lambda i: (0, i, 0)),
            pl.BlockSpec((D, D), lambda i: (0, 0)),
            pl.BlockSpec((DE, D), lambda i: (0, 0)),
            pl.BlockSpec((1, D), lambda i: (0, 0)),
        ],
        out_specs=(
            pl.BlockSpec((NB, D), lambda i: (i, 0)),
            pl.BlockSpec((1, D), lambda i: (0, 0)),
            pl.BlockSpec((1, DE), lambda i: (0, 0)),
        ),
        out_shape=(
            jax.ShapeDtypeStruct((N, D), jnp.float32),
            jax.ShapeDtypeStruct((1, D), jnp.float32),
            jax.ShapeDtypeStruct((1, DE), jnp.float32),
        ),
    )(node_attr, aggp, W_v1, W_v2, b_v.reshape(1, D))

    u_out = pl.pallas_call(
        _global_body,
        out_shape=jax.ShapeDtypeStruct((1, DU), jnp.float32),
    )(vsum, esum, u, W_uv, W_ue, W_uu, b_u.reshape(1, DU))

    return (v_out, e_out, u_out)


# speedup vs baseline: 7.8608x; 1.3262x over previous
import functools

import jax
import jax.numpy as jnp
from jax import lax
from jax.experimental import pallas as pl
from jax.experimental.pallas import tpu as pltpu
from jax.experimental.pallas import tpu_sc as plsc

N = 10000
E = 320000
D = 128
DE = 16
DU = 32

NC = 2
NS = 16
NW = NC * NS
CH = 640
SUB = 128
NSUB = CH // SUB
NCHUNK = E // CH
EP = E // 8
CHP = CH // 8
N_PAD = 10240
ROWS_PER_TILE = N_PAD // NS


_sc_mesh = plsc.VectorSubcoreMesh(
    core_axis_name="c", subcore_axis_name="s", num_cores=NC, num_subcores=NS
)


@functools.partial(
    pl.kernel,
    out_type=(
        jax.ShapeDtypeStruct((EP, 128), jnp.float32),
        jax.ShapeDtypeStruct((NC, N_PAD, DE), jnp.float32),
    ),
    mesh=_sc_mesh,
    scratch_types=[
        pltpu.VMEM((NSUB, SUB), jnp.int32),
        pltpu.VMEM((NSUB, SUB), jnp.int32),
        pltpu.VMEM((CHP, 128), jnp.float32),
        pltpu.VMEM((CH, DE), jnp.float32),
        pltpu.VMEM((CH, DE), jnp.float32),
        pltpu.VMEM((CH, DE), jnp.float32),
        pltpu.VMEM((CHP, 128), jnp.float32),
        pltpu.VMEM((ROWS_PER_TILE, DE), jnp.float32),
        pltpu.VMEM_SHARED((N_PAD, DE), jnp.float32),
        pltpu.SemaphoreType.DMA,
        pltpu.SemaphoreType.DMA,
    ],
    compiler_params=pltpu.CompilerParams(use_tc_tiling_on_sc=False),
)
def _edge_sc(row_hbm, col_hbm, et_hbm, psrc_hbm, pdst_hbm,
             eout_hbm, aggp_hbm,
             idxr_v, idxc_v, et_v, src_v, dst_v, eo_v, eo40_v, z_v, agg_sh,
             sem1, sem2):
    cid = lax.axis_index("c")
    sid = lax.axis_index("s")
    wid = sid * NC + cid

    def zero_body(i, _):
        z_v[i] = jnp.zeros((DE,), jnp.float32)
        return 0
    lax.fori_loop(0, ROWS_PER_TILE, zero_body, 0)
    pltpu.sync_copy(z_v, agg_sh.at[pl.ds(sid * ROWS_PER_TILE, ROWS_PER_TILE)])
    plsc.subcore_barrier()

    base_chunks = NCHUNK // NW
    nloc = jnp.where(wid < NCHUNK - base_chunks * NW, base_chunks + 1,
                     base_chunks)

    def chunk_body(j, _):
        chunk = wid + j * NW
        cpr = pltpu.async_copy(row_hbm.at[pl.ds(chunk * NSUB, NSUB)], idxr_v,
                               sem2)
        cpc = pltpu.async_copy(col_hbm.at[pl.ds(chunk * NSUB, NSUB)], idxc_v,
                               sem2)
        cpe = pltpu.async_copy(et_hbm.at[pl.ds(chunk * CHP, CHP)], et_v, sem2)
        cpr.wait()
        cpc.wait()
        gathers = []
        for k in range(NSUB):
            gathers.append(pltpu.async_copy(
                psrc_hbm.at[idxr_v.at[k]], src_v.at[pl.ds(k * SUB, SUB)],
                sem1))
            gathers.append(pltpu.async_copy(
                pdst_hbm.at[idxc_v.at[k]], dst_v.at[pl.ds(k * SUB, SUB)],
                sem1))
        for g in gathers:
            g.wait()
        cpe.wait()

        def edge_body(i, _):
            t = i // 8
            o = (i % 8) * DE
            val = jnp.maximum(src_v[i] + dst_v[i] + et_v[t, pl.ds(o, DE)], 0.0)
            eo_v[i] = val
            eo40_v[t, pl.ds(o, DE)] = val
            return 0
        lax.fori_loop(0, CH, edge_body, 0)

        pltpu.sync_copy(eo40_v, eout_hbm.at[pl.ds(chunk * CHP, CHP)])
        for k in range(NSUB):
            pltpu.sync_copy(
                eo_v.at[pl.ds(k * SUB, SUB)],
                agg_sh.at[idxc_v.at[k]], add=True)
        return 0

    lax.fori_loop(0, nloc, chunk_body, 0)

    plsc.subcore_barrier()
    pltpu.sync_copy(
        agg_sh.at[pl.ds(sid * ROWS_PER_TILE, ROWS_PER_TILE)],
        aggp_hbm.at[cid, pl.ds(sid * ROWS_PER_TILE, ROWS_PER_TILE)],
    )


def _proj_body(node_ref, wsrc_ref, wdst_ref, psrc_ref, pdst_ref):
    x = node_ref[...]
    psrc_ref[...] = jnp.dot(x, wsrc_ref[...], preferred_element_type=jnp.float32)
    pdst_ref[...] = jnp.dot(x, wdst_ref[...], preferred_element_type=jnp.float32)


def _edge_term_body(ea_ref, w_ref, b_ref, out_ref):
    out_ref[...] = (
        jnp.dot(ea_ref[...], w_ref[...], preferred_element_type=jnp.float32)
        + b_ref[...]
    )


def _node_body(node_ref, aggp_ref, wv1_ref, wv2_ref, bv_ref,
               vout_ref, vsum_ref, esum_ref):
    i = pl.program_id(0)
    agg = aggp_ref[0] + aggp_ref[1]
    v = jnp.dot(node_ref[...], wv1_ref[...], preferred_element_type=jnp.float32)
    v += jnp.dot(agg, wv2_ref[...], preferred_element_type=jnp.float32)
    v = jnp.maximum(v + bv_ref[...], 0.0)
    vout_ref[...] = v

    @pl.when(i == 0)
    def _():
        vsum_ref[...] = jnp.zeros_like(vsum_ref)
        esum_ref[...] = jnp.zeros_like(esum_ref)
    vsum_ref[...] += jnp.sum(v, axis=0, keepdims=True)
    esum_ref[...] += jnp.sum(agg, axis=0, keepdims=True)


def _global_body(vsum_ref, esum_ref, u_ref, wuv_ref, wue_ref, wuu_ref, bu_ref,
                 out_ref):
    mean_v = vsum_ref[...] / N
    mean_e = esum_ref[...] / E
    acc = jnp.dot(mean_v, wuv_ref[...], preferred_element_type=jnp.float32)
    acc += jnp.dot(mean_e, wue_ref[...], preferred_element_type=jnp.float32)
    acc += jnp.dot(u_ref[...], wuu_ref[...], preferred_element_type=jnp.float32)
    out_ref[...] = jnp.maximum(acc + bu_ref[...], 0.0)


def kernel(node_attr, connectivity, edge_attr, u, W_e, b_e, W_v, b_v, W_u, b_u):
    row = connectivity[0]
    col = connectivity[1]
    row2d = row.reshape(E // SUB, SUB)
    col2d = col.reshape(E // SUB, SUB)
    W_src = W_e[:D]
    W_dst = W_e[D:2 * D]
    W_eg = W_e[2 * D:]
    W_v1 = W_v[:D]
    W_v2 = W_v[D:]
    W_uv = W_u[:D]
    W_ue = W_u[D:D + DE]
    W_uu = W_u[D + DE:]

    eye8 = jnp.eye(8, dtype=jnp.float32)
    psrc_p, pdst_p = pl.pallas_call(
        _proj_body,
        out_shape=(
            jax.ShapeDtypeStruct((N // 8, 128), jnp.float32),
            jax.ShapeDtypeStruct((N // 8, 128), jnp.float32),
        ),
    )(node_attr.reshape(N // 8, 8 * D), jnp.kron(eye8, W_src),
      jnp.kron(eye8, W_dst))
    psrc = psrc_p.reshape(N, DE)
    pdst = pdst_p.reshape(N, DE)

    ea40 = (
        edge_attr.T.reshape(DE, EP, 8).transpose(1, 2, 0).reshape(EP, 128)
    )
    W_blk = jnp.kron(eye8, W_eg)
    b_blk = jnp.tile(b_e, 8).reshape(1, 128)
    EBP = 2000
    edge_term = pl.pallas_call(
        _edge_term_body,
        grid=(EP // EBP,),
        in_specs=[
            pl.BlockSpec((EBP, 128), lambda i: (i, 0)),
            pl.BlockSpec((128, 128), lambda i: (0, 0)),
            pl.BlockSpec((1, 128), lambda i: (0, 0)),
        ],
        out_specs=pl.BlockSpec((EBP, 128), lambda i: (i, 0)),
        out_shape=jax.ShapeDtypeStruct((EP, 128), jnp.float32),
    )(ea40, W_blk, b_blk)

    eout_packed, aggp = _edge_sc(row2d, col2d, edge_term, psrc, pdst)
    e_out = (
        eout_packed.reshape(EP, 8, DE).transpose(2, 0, 1).reshape(DE, E).T
    )

    NB = 2000
    v_out, vsum, esum = pl.pallas_call(
        _node_body,
        grid=(N // NB,),
        in_specs=[
            pl.BlockSpec((NB, D), lambda i: (i, 0)),
            pl.BlockSpec((NC, NB, DE), lambda i: (0, i, 0)),
            pl.BlockSpec((D, D), lambda i: (0, 0)),
            pl.BlockSpec((DE, D), lambda i: (0, 0)),
            pl.BlockSpec((1, D), lambda i: (0, 0)),
        ],
        out_specs=(
            pl.BlockSpec((NB, D), lambda i: (i, 0)),
            pl.BlockSpec((1, D), lambda i: (0, 0)),
            pl.BlockSpec((1, DE), lambda i: (0, 0)),
        ),
        out_shape=(
            jax.ShapeDtypeStruct((N, D), jnp.float32),
            jax.ShapeDtypeStruct((1, D), jnp.float32),
            jax.ShapeDtypeStruct((1, DE), jnp.float32),
        ),
    )(node_attr, aggp, W_v1, W_v2, b_v.reshape(1, D))

    u_out = pl.pallas_call(
        _global_body,
        out_shape=jax.ShapeDtypeStruct((1, DU), jnp.float32),
    )(vsum, esum, u, W_uv, W_ue, W_uu, b_u.reshape(1, DU))

    return (v_out, e_out, u_out)
